# Initial kernel scaffold; baseline (speedup 1.0000x reference)
import jax
import jax.numpy as jnp
from jax.experimental import pallas as pl


def kernel(x, src_mask, src_lengths, durations, pitch, energy, max_length, params, p_control, e_control, d_control):
    raise NotImplementedError("write your pallas kernel here")


# speedup vs baseline: 16.8517x; 16.8517x over previous
import functools

import jax
import jax.numpy as jnp
from jax import lax
from jax.experimental import pallas as pl
from jax.experimental.pallas import tpu as pltpu
from jax.experimental.pallas import tpu_sc as plsc

_B, _T, _D = 16, 512, 256
_FS, _K, _NB = 256, 3, 256
_MAXLEN = 1500
_PAD_A = 1504
_CH = 125
_NCH = 6
_HALF = _MAXLEN // 2
_ZROW = _B * _T


def _shift_dn(a):
    r = pltpu.roll(a, 1, 0)
    rows = lax.broadcasted_iota(jnp.int32, a.shape, 0)
    return jnp.where(rows == 0, 0.0, r)


def _shift_up(a):
    r = pltpu.roll(a, a.shape[0] - 1, 0)
    rows = lax.broadcasted_iota(jnp.int32, a.shape, 0)
    return jnp.where(rows == a.shape[0] - 1, 0.0, r)


def _conv_block(h, w_ref, b_ref):
    cat = jnp.concatenate([_shift_dn(h), h, _shift_up(h)], axis=1)
    o = jnp.dot(cat, w_ref[...], preferred_element_type=jnp.float32)
    return jnp.maximum(o + b_ref[0], 0.0)


def _emb_lookup(vals_col, bins_ref, tab_ref):
    cmp = (bins_ref[0:1, :] < vals_col).astype(jnp.int32)
    idx = jnp.sum(cmp, axis=1, keepdims=True)
    oh = (idx == lax.broadcasted_iota(jnp.int32, (_T, _NB), 1)).astype(jnp.float32)
    return jnp.dot(oh, tab_ref[...], preferred_element_type=jnp.float32)


def _tc_body(x_ref, pv_ref, ev_ref, mk_ref, pbins_ref, ebins_ref, ptab_ref, etab_ref,
             w1d_ref, b1d_ref, w2d_ref, b2d_ref, wpd_ref,
             w1p_ref, b1p_ref, w2p_ref, b2p_ref, wpp_ref,
             w1e_ref, b1e_ref, w2e_ref, b2e_ref, wpe_ref,
             ld_ref, pp_ref, ep_ref, x2_ref):
    b = pl.program_id(0)

    @pl.when(b == _B)
    def _():
        ld_ref[...] = jnp.zeros_like(ld_ref)
        pp_ref[...] = jnp.zeros_like(pp_ref)
        ep_ref[...] = jnp.zeros_like(ep_ref)
        x2_ref[...] = jnp.zeros_like(x2_ref)

    @pl.when(b < _B)
    def _():
        x = x_ref[0]
        inv = 1.0 - mk_ref[0]

        def predictor(h0, w1_ref, b1_ref, w2_ref, b2_ref, wp_ref):
            h1 = _conv_block(h0, w1_ref, b1_ref)
            h2 = _conv_block(h1, w2_ref, b2_ref)
            o = jnp.sum(h2 * wp_ref[0:1, :], axis=1, keepdims=True) + wp_ref[1, 0]
            return o * inv

        p_emb = _emb_lookup(pv_ref[0], pbins_ref, ptab_ref)
        x1 = x + p_emb
        e_emb = _emb_lookup(ev_ref[0], ebins_ref, etab_ref)
        x2_ref[0] = x1 + e_emb

        ld_ref[0] = predictor(x, w1d_ref, b1d_ref, w2d_ref, b2d_ref, wpd_ref)
        pp_ref[0] = predictor(x, w1p_ref, b1p_ref, w2p_ref, b2p_ref, wpp_ref)
        ep_ref[0] = predictor(x1, w1e_ref, b1e_ref, w2e_ref, b2e_ref, wpe_ref)


def _batch_spec(shape):
    return pl.BlockSpec((1,) + shape, lambda i: (jnp.minimum(i, _B - 1), 0, 0))


def _full_spec(shape):
    zeros = (0,) * len(shape)
    return pl.BlockSpec(shape, lambda i, _z=zeros: _z)


def _tc_call(x, pv, ev, mk, pbins, ebins, ptab, etab, wd, wp_, we):
    f32 = jnp.float32
    in_specs = [
        _batch_spec((_T, _D)),
        _batch_spec((_T, 1)),
        _batch_spec((_T, 1)),
        _batch_spec((_T, 1)),
        _full_spec((1, _NB)),
        _full_spec((1, _NB)),
        _full_spec((_NB, _D)),
        _full_spec((_NB, _D)),
    ]
    args = [x, pv, ev, mk, pbins, ebins, ptab, etab]
    for w1, b1, w2, b2, wpb in (wd, wp_, we):
        in_specs += [_full_spec((3 * _D, _FS)), _full_spec((1, _FS)),
                     _full_spec((3 * _FS, _FS)), _full_spec((1, _FS)),
                     _full_spec((2, _FS))]
        args += [w1, b1, w2, b2, wpb]
    out_specs = [
        pl.BlockSpec((1, _T, 1), lambda i: (i, 0, 0)),
        pl.BlockSpec((1, _T, 1), lambda i: (i, 0, 0)),
        pl.BlockSpec((1, _T, 1), lambda i: (i, 0, 0)),
        pl.BlockSpec((1, _T, _D), lambda i: (i, 0, 0)),
    ]
    out_shape = [
        jax.ShapeDtypeStruct((_B + 1, _T, 1), f32),
        jax.ShapeDtypeStruct((_B + 1, _T, 1), f32),
        jax.ShapeDtypeStruct((_B + 1, _T, 1), f32),
        jax.ShapeDtypeStruct((_B + 1, _T, _D), f32),
    ]
    return pl.pallas_call(
        _tc_body,
        grid=(_B + 1,),
        in_specs=in_specs,
        out_specs=out_specs,
        out_shape=out_shape,
    )(*args)


def _sc_body(dur_hbm, x2_hbm, out_hbm, lens_hbm, dur_v, a_v, gidx_v, buf_v, lens_v, sem):
    c = lax.axis_index("c")
    s = lax.axis_index("s")
    wid = s * 2 + c
    b = wid // 2
    h = wid % 2

    pltpu.sync_copy(dur_hbm.at[b], dur_v)

    iota = lax.broadcasted_iota(jnp.int32, (16,), 0)
    zeros16 = jnp.zeros((16,), jnp.int32)

    def zbody(j, carry):
        a_v[pl.ds(j * 16, 16)] = zeros16
        return carry
    lax.fori_loop(0, _PAD_A // 16, zbody, jnp.int32(0))

    def cbody(i, carry):
        d = dur_v[0, pl.ds(i * 16, 16)]
        cum = plsc.cumsum(d) + carry
        prev = cum - d
        m = (d > 0) & (prev < _PAD_A)
        plsc.store_scatter(a_v, [prev], iota + i * 16, mask=m)
        return jnp.max(cum)
    total = lax.fori_loop(0, _T // 16, cbody, jnp.int32(0))
    len_b = jnp.minimum(total, _MAXLEN)

    @pl.when(h == 0)
    def _():
        lens_v[0, :] = jnp.full((16,), total, jnp.int32)
        pltpu.sync_copy(lens_v, lens_hbm.at[b])

    def fbody(j, carry):
        m = jnp.maximum(plsc.cummax(a_v[pl.ds(j * 16, 16)]), carry)
        a_v[pl.ds(j * 16, 16)] = m
        return jnp.max(m)
    lax.fori_loop(0, _PAD_A // 16, fbody, jnp.int32(0))

    base = h * _HALF
    for k in range(_NCH):
        t0 = base + k * _CH
        for j in range(8):
            t = t0 + j * 16
            a = a_v[pl.ds(t, 16)]
            g = jnp.where(iota + t < len_b, b * _T + a, _ZROW)
            gidx_v[pl.ds(j * 16, 16)] = g
        pltpu.async_copy(x2_hbm.at[gidx_v], buf_v, sem).wait()
        pltpu.sync_copy(buf_v.at[pl.ds(0, _CH)], out_hbm.at[b, pl.ds(t0, _CH)])


@functools.cache
def _sc_call():
    return pl.kernel(
        _sc_body,
        out_type=[jax.ShapeDtypeStruct((_B, _MAXLEN, 2, 128), jnp.float32),
                  jax.ShapeDtypeStruct((_B, 1, 16), jnp.int32)],
        mesh=plsc.VectorSubcoreMesh(core_axis_name="c", subcore_axis_name="s",
                                    num_cores=2, num_subcores=16),
        scratch_types=[pltpu.VMEM((1, _T), jnp.int32),
                       pltpu.VMEM((_PAD_A,), jnp.int32),
                       pltpu.VMEM((128,), jnp.int32),
                       pltpu.VMEM((128, 2, 128), jnp.float32),
                       pltpu.VMEM((1, 16), jnp.int32),
                       pltpu.SemaphoreType.DMA],
        compiler_params=pltpu.CompilerParams(needs_layout_passes=False),
    )


def _fold_bn(p):
    s1 = p['g1'] / jnp.sqrt(p['v1'] + 1e-5)
    s2 = p['g2'] / jnp.sqrt(p['v2'] + 1e-5)
    w1 = p['w1'] * s1[:, None, None]
    w2 = p['w2'] * s2[:, None, None]
    W1 = jnp.concatenate([w1[:, :, 0].T, w1[:, :, 1].T, w1[:, :, 2].T], axis=0)
    W2 = jnp.concatenate([w2[:, :, 0].T, w2[:, :, 1].T, w2[:, :, 2].T], axis=0)
    b1 = ((p['b1'] - p['m1']) * s1 + p['be1'])[None, :]
    b2 = ((p['b2'] - p['m2']) * s2 + p['be2'])[None, :]
    wpb = jnp.concatenate([p['wp'], jnp.full((1, _FS), p['bp'][0])], axis=0)
    return W1, b1, W2, b2, wpb


def _pad_bins(bins):
    return jnp.concatenate([bins, jnp.full((1,), 1e30, jnp.float32)])[None, :]


def kernel(x, src_mask, src_lengths, durations, pitch, energy, max_length,
           params, p_control=1.0, e_control=1.0, d_control=1.0):
    f32 = jnp.float32
    pv = pitch[..., None].astype(f32)
    ev = energy[..., None].astype(f32)
    mk = src_mask[..., None].astype(f32)
    wd = _fold_bn(params['dur'])
    wp_ = _fold_bn(params['pitch'])
    we = _fold_bn(params['energy'])

    ld_pad, pp_pad, ep_pad, x2_pad = _tc_call(
        x, pv, ev, mk,
        _pad_bins(params['pitch_bins']), _pad_bins(params['energy_bins']),
        params['pitch_table'], params['energy_table'], wd, wp_, we)

    dur_rounded = jnp.round(durations.astype(f32) * d_control).astype(jnp.int32)
    out4, lens16 = _sc_call()(dur_rounded.reshape(_B, 1, _T),
                              x2_pad.reshape((_B + 1) * _T, 2, 128))
    out = out4.reshape(_B, _MAXLEN, _D)

    log_d = ld_pad[:_B, :, 0]
    p_pred = pp_pad[:_B, :, 0]
    e_pred = ep_pad[:_B, :, 0]
    mel_lens = jnp.minimum(lens16[:, 0, 0], max_length)
    mel_mask = jnp.arange(_MAXLEN)[None, :] >= mel_lens[:, None]
    return out, log_d, p_pred, e_pred, mel_lens, mel_mask


# speedup vs baseline: 17.3954x; 1.0323x over previous
import functools

import jax
import jax.numpy as jnp
from jax import lax
from jax.experimental import pallas as pl
from jax.experimental.pallas import tpu as pltpu
from jax.experimental.pallas import tpu_sc as plsc

_B, _T, _D = 16, 512, 256
_FS, _K, _NB = 256, 3, 256
_MAXLEN = 1500
_PAD_A = 1504
_CH = 125
_NCH = 6
_HALF = _MAXLEN // 2
_ZROW = _B * _T


def _shift_dn(a):
    r = pltpu.roll(a, 1, 0)
    rows = lax.broadcasted_iota(jnp.int32, a.shape, 0)
    return jnp.where(rows == 0, 0.0, r)


def _shift_up(a):
    r = pltpu.roll(a, a.shape[0] - 1, 0)
    rows = lax.broadcasted_iota(jnp.int32, a.shape, 0)
    return jnp.where(rows == a.shape[0] - 1, 0.0, r)


def _conv_block(h, w_ref, b_ref):
    cat = jnp.concatenate([_shift_dn(h), h, _shift_up(h)], axis=1)
    o = jnp.dot(cat, w_ref[...], preferred_element_type=jnp.float32)
    return jnp.maximum(o + b_ref[0], 0.0)


def _emb_lookup(vals_col, bins_ref, tab_ref):
    cmp = (bins_ref[0:1, :] < vals_col).astype(jnp.int32)
    idx = jnp.sum(cmp, axis=1, keepdims=True)
    oh = (idx == lax.broadcasted_iota(jnp.int32, (_T, _NB), 1)).astype(jnp.float32)
    return jnp.dot(oh, tab_ref[...], preferred_element_type=jnp.float32)


def _emb_body(x_ref, pv_ref, ev_ref, pbins_ref, ebins_ref, ptab_ref, etab_ref,
              x1_ref, x2_ref):
    b = pl.program_id(0)
    x = x_ref[0]
    p_emb = _emb_lookup(pv_ref[0], pbins_ref, ptab_ref)
    x1 = x + p_emb
    e_emb = _emb_lookup(ev_ref[0], ebins_ref, etab_ref)
    x1_ref[0] = x1
    zf = jnp.where(b < _B, 1.0, 0.0)
    x2 = (x1 + e_emb) * zf
    x2_ref[0, :, 0, :] = x2[:, :128]
    x2_ref[0, :, 1, :] = x2[:, 128:]


def _pred_body(x_ref, x1_ref, mk_ref,
               w1d_ref, b1d_ref, w2d_ref, b2d_ref, wpd_ref,
               w1p_ref, b1p_ref, w2p_ref, b2p_ref, wpp_ref,
               w1e_ref, b1e_ref, w2e_ref, b2e_ref, wpe_ref,
               ld_ref, pp_ref, ep_ref):
    x = x_ref[0]
    inv = 1.0 - mk_ref[0]

    def predictor(h0, w1_ref, b1_ref, w2_ref, b2_ref, wp_ref):
        h1 = _conv_block(h0, w1_ref, b1_ref)
        h2 = _conv_block(h1, w2_ref, b2_ref)
        o = jnp.sum(h2 * wp_ref[0:1, :], axis=1, keepdims=True) + wp_ref[1, 0]
        return o * inv

    ld_ref[0] = predictor(x, w1d_ref, b1d_ref, w2d_ref, b2d_ref, wpd_ref)
    pp_ref[0] = predictor(x, w1p_ref, b1p_ref, w2p_ref, b2p_ref, wpp_ref)
    ep_ref[0] = predictor(x1_ref[0], w1e_ref, b1e_ref, w2e_ref, b2e_ref, wpe_ref)


def _batch_spec(shape):
    return pl.BlockSpec((1,) + shape, lambda i: (jnp.minimum(i, _B - 1), 0, 0))


def _full_spec(shape):
    zeros = (0,) * len(shape)
    return pl.BlockSpec(shape, lambda i, _z=zeros: _z)


def _emb_call(x, pv, ev, pbins, ebins, ptab, etab):
    f32 = jnp.float32
    in_specs = [
        _batch_spec((_T, _D)),
        _batch_spec((_T, 1)),
        _batch_spec((_T, 1)),
        _full_spec((1, _NB)),
        _full_spec((1, _NB)),
        _full_spec((_NB, _D)),
        _full_spec((_NB, _D)),
    ]
    out_specs = [
        pl.BlockSpec((1, _T, _D), lambda i: (jnp.minimum(i, _B - 1), 0, 0)),
        pl.BlockSpec((1, _T, 2, 128), lambda i: (i, 0, 0, 0)),
    ]
    out_shape = [
        jax.ShapeDtypeStruct((_B, _T, _D), f32),
        jax.ShapeDtypeStruct((_B + 1, _T, 2, 128), f32),
    ]
    return pl.pallas_call(
        _emb_body,
        grid=(_B + 1,),
        in_specs=in_specs,
        out_specs=out_specs,
        out_shape=out_shape,
    )(x, pv, ev, pbins, ebins, ptab, etab)


def _pred_call(x, x1, mk, wd, wp_, we):
    f32 = jnp.float32
    in_specs = [
        _batch_spec((_T, _D)),
        _batch_spec((_T, _D)),
        _batch_spec((_T, 1)),
    ]
    args = [x, x1, mk]
    for w1, b1, w2, b2, wpb in (wd, wp_, we):
        in_specs += [_full_spec((3 * _D, _FS)), _full_spec((1, _FS)),
                     _full_spec((3 * _FS, _FS)), _full_spec((1, _FS)),
                     _full_spec((2, _FS))]
        args += [w1, b1, w2, b2, wpb]
    out_specs = [
        pl.BlockSpec((1, _T, 1), lambda i: (i, 0, 0)),
        pl.BlockSpec((1, _T, 1), lambda i: (i, 0, 0)),
        pl.BlockSpec((1, _T, 1), lambda i: (i, 0, 0)),
    ]
    out_shape = [
        jax.ShapeDtypeStruct((_B, _T, 1), f32),
        jax.ShapeDtypeStruct((_B, _T, 1), f32),
        jax.ShapeDtypeStruct((_B, _T, 1), f32),
    ]
    return pl.pallas_call(
        _pred_body,
        grid=(_B,),
        in_specs=in_specs,
        out_specs=out_specs,
        out_shape=out_shape,
    )(*args)


def _sc_body(dur_hbm, x2_hbm, out_hbm, lens_hbm, dur_v, a_v, gidx_v, buf_v, lens_v, sem):
    c = lax.axis_index("c")
    s = lax.axis_index("s")
    wid = s * 2 + c
    b = wid // 2
    h = wid % 2

    pltpu.sync_copy(dur_hbm.at[b], dur_v)

    iota = lax.broadcasted_iota(jnp.int32, (16,), 0)
    zeros16 = jnp.zeros((16,), jnp.int32)

    def zbody(j, carry):
        a_v[pl.ds(j * 16, 16)] = zeros16
        return carry
    lax.fori_loop(0, _PAD_A // 16, zbody, jnp.int32(0))

    def cbody(i, carry):
        d = dur_v[0, pl.ds(i * 16, 16)]
        cum = plsc.cumsum(d) + carry
        prev = cum - d
        m = (d > 0) & (prev < _PAD_A)
        plsc.store_scatter(a_v, [prev], iota + i * 16, mask=m)
        return jnp.max(cum)
    total = lax.fori_loop(0, _T // 16, cbody, jnp.int32(0))
    len_b = jnp.minimum(total, _MAXLEN)

    @pl.when(h == 0)
    def _():
        lens_v[0, :] = jnp.full((16,), total, jnp.int32)
        pltpu.sync_copy(lens_v, lens_hbm.at[b])

    def fbody(j, carry):
        m = jnp.maximum(plsc.cummax(a_v[pl.ds(j * 16, 16)]), carry)
        a_v[pl.ds(j * 16, 16)] = m
        return jnp.max(m)
    lax.fori_loop(0, _PAD_A // 16, fbody, jnp.int32(0))

    base = h * _HALF
    for k in range(_NCH):
        t0 = base + k * _CH
        for j in range(8):
            t = t0 + j * 16
            a = a_v[pl.ds(t, 16)]
            g = jnp.where(iota + t < len_b, b * _T + a, _ZROW)
            gidx_v[pl.ds(j * 16, 16)] = g
        pltpu.async_copy(x2_hbm.at[gidx_v], buf_v, sem).wait()
        pltpu.sync_copy(buf_v.at[pl.ds(0, _CH)], out_hbm.at[b, pl.ds(t0, _CH)])


@functools.cache
def _sc_call():
    return pl.kernel(
        _sc_body,
        out_type=[jax.ShapeDtypeStruct((_B, _MAXLEN, 2, 128), jnp.float32),
                  jax.ShapeDtypeStruct((_B, 1, 16), jnp.int32)],
        mesh=plsc.VectorSubcoreMesh(core_axis_name="c", subcore_axis_name="s",
                                    num_cores=2, num_subcores=16),
        scratch_types=[pltpu.VMEM((1, _T), jnp.int32),
                       pltpu.VMEM((_PAD_A,), jnp.int32),
                       pltpu.VMEM((128,), jnp.int32),
                       pltpu.VMEM((128, 2, 128), jnp.float32),
                       pltpu.VMEM((1, 16), jnp.int32),
                       pltpu.SemaphoreType.DMA],
        compiler_params=pltpu.CompilerParams(needs_layout_passes=False),
    )


def _fold_bn(p):
    s1 = p['g1'] / jnp.sqrt(p['v1'] + 1e-5)
    s2 = p['g2'] / jnp.sqrt(p['v2'] + 1e-5)
    w1 = p['w1'] * s1[:, None, None]
    w2 = p['w2'] * s2[:, None, None]
    W1 = jnp.concatenate([w1[:, :, 0].T, w1[:, :, 1].T, w1[:, :, 2].T], axis=0)
    W2 = jnp.concatenate([w2[:, :, 0].T, w2[:, :, 1].T, w2[:, :, 2].T], axis=0)
    b1 = ((p['b1'] - p['m1']) * s1 + p['be1'])[None, :]
    b2 = ((p['b2'] - p['m2']) * s2 + p['be2'])[None, :]
    wpb = jnp.concatenate([p['wp'], jnp.full((1, _FS), p['bp'][0])], axis=0)
    return W1, b1, W2, b2, wpb


def _pad_bins(bins):
    return jnp.concatenate([bins, jnp.full((1,), 1e30, jnp.float32)])[None, :]


def kernel(x, src_mask, src_lengths, durations, pitch, energy, max_length,
           params, p_control=1.0, e_control=1.0, d_control=1.0):
    f32 = jnp.float32
    pv = pitch[..., None].astype(f32)
    ev = energy[..., None].astype(f32)
    mk = src_mask[..., None].astype(f32)
    wd = _fold_bn(params['dur'])
    wp_ = _fold_bn(params['pitch'])
    we = _fold_bn(params['energy'])

    x1, x2_pad = _emb_call(
        x, pv, ev,
        _pad_bins(params['pitch_bins']), _pad_bins(params['energy_bins']),
        params['pitch_table'], params['energy_table'])

    dur_rounded = jnp.round(durations.astype(f32) * d_control).astype(jnp.int32)
    out4, lens16 = _sc_call()(dur_rounded.reshape(_B, 1, _T),
                              x2_pad.reshape((_B + 1) * _T, 2, 128))
    out = out4.reshape(_B, _MAXLEN, _D)

    ld_pad, pp_pad, ep_pad = _pred_call(x, x1, mk, wd, wp_, we)
    log_d = ld_pad[:, :, 0]
    p_pred = pp_pad[:, :, 0]
    e_pred = ep_pad[:, :, 0]
    mel_lens = jnp.minimum(lens16[:, 0, 0], max_length)
    mel_mask = jnp.arange(_MAXLEN)[None, :] >= mel_lens[:, None]
    return out, log_d, p_pred, e_pred, mel_lens, mel_mask


# speedup vs baseline: 21.1230x; 1.2143x over previous
import functools

import jax
import jax.numpy as jnp
from jax import lax
from jax.experimental import pallas as pl
from jax.experimental.pallas import tpu as pltpu
from jax.experimental.pallas import tpu_sc as plsc

_B, _T, _D = 16, 512, 256
_FS, _K, _NB = 256, 3, 256
_MAXLEN = 1500
_PAD_A = 1504
_ZROW = _B * _T
_CHUNKS = (
    ((0, 128), (128, 128), (256, 128), (384, 128), (512, 128), (640, 128)),
    ((768, 128), (896, 128), (1024, 128), (1152, 128), (1280, 128), (1408, 88)),
)


def _shift_dn(a):
    r = pltpu.roll(a, 1, 0)
    rows = lax.broadcasted_iota(jnp.int32, a.shape, 0)
    return jnp.where(rows == 0, 0.0, r)


def _shift_up(a):
    r = pltpu.roll(a, a.shape[0] - 1, 0)
    rows = lax.broadcasted_iota(jnp.int32, a.shape, 0)
    return jnp.where(rows == a.shape[0] - 1, 0.0, r)


def _conv_block(h, w_ref, b_ref):
    cat = jnp.concatenate([_shift_dn(h), h, _shift_up(h)], axis=1)
    o = jnp.dot(cat, w_ref[...], preferred_element_type=jnp.float32)
    return jnp.maximum(o + b_ref[0], 0.0)


def _emb_lookup(vals_col, bins_ref, tab_ref):
    cmp = (bins_ref[0:1, :] < vals_col).astype(jnp.int32)
    idx = jnp.sum(cmp, axis=1, keepdims=True)
    oh = (idx == lax.broadcasted_iota(jnp.int32, (_T, _NB), 1)).astype(jnp.float32)
    return jnp.dot(oh, tab_ref[...], preferred_element_type=jnp.float32)


def _emb_body(x_ref, pv_ref, ev_ref, pbins_ref, ebins_ref, ptab_ref, etab_ref,
              x1_ref, x2_ref):
    b = pl.program_id(0)
    x = x_ref[0]
    p_emb = _emb_lookup(pv_ref[0], pbins_ref, ptab_ref)
    x1 = x + p_emb
    e_emb = _emb_lookup(ev_ref[0], ebins_ref, etab_ref)
    x1_ref[0] = x1
    zf = jnp.where(b < _B, 1.0, 0.0)
    x2_ref[...] = (x1 + e_emb) * zf


def _pred_body(x_ref, x1_ref, mk_ref,
               w1d_ref, b1d_ref, w2d_ref, b2d_ref, wpd_ref,
               w1p_ref, b1p_ref, w2p_ref, b2p_ref, wpp_ref,
               w1e_ref, b1e_ref, w2e_ref, b2e_ref, wpe_ref,
               ld_ref, pp_ref, ep_ref):
    x = x_ref[0]
    inv = 1.0 - mk_ref[0]

    def predictor(h0, w1_ref, b1_ref, w2_ref, b2_ref, wp_ref):
        h1 = _conv_block(h0, w1_ref, b1_ref)
        h2 = _conv_block(h1, w2_ref, b2_ref)
        o = jnp.sum(h2 * wp_ref[0:1, :], axis=1, keepdims=True) + wp_ref[1, 0]
        return o * inv

    ld_ref[0] = predictor(x, w1d_ref, b1d_ref, w2d_ref, b2d_ref, wpd_ref)
    pp_ref[0] = predictor(x, w1p_ref, b1p_ref, w2p_ref, b2p_ref, wpp_ref)
    ep_ref[0] = predictor(x1_ref[0], w1e_ref, b1e_ref, w2e_ref, b2e_ref, wpe_ref)


def _batch_spec(shape):
    return pl.BlockSpec((1,) + shape, lambda i: (jnp.minimum(i, _B - 1), 0, 0))


def _full_spec(shape):
    zeros = (0,) * len(shape)
    return pl.BlockSpec(shape, lambda i, _z=zeros: _z)


def _emb_call(x, pv, ev, pbins, ebins, ptab, etab):
    f32 = jnp.float32
    in_specs = [
        _batch_spec((_T, _D)),
        _batch_spec((_T, 1)),
        _batch_spec((_T, 1)),
        _full_spec((1, _NB)),
        _full_spec((1, _NB)),
        _full_spec((_NB, _D)),
        _full_spec((_NB, _D)),
    ]
    out_specs = [
        pl.BlockSpec((1, _T, _D), lambda i: (jnp.minimum(i, _B - 1), 0, 0)),
        pl.BlockSpec((_T, _D), lambda i: (i, 0)),
    ]
    out_shape = [
        jax.ShapeDtypeStruct((_B, _T, _D), f32),
        jax.ShapeDtypeStruct(((_B + 1) * _T, _D), f32),
    ]
    return pl.pallas_call(
        _emb_body,
        grid=(_B + 1,),
        in_specs=in_specs,
        out_specs=out_specs,
        out_shape=out_shape,
    )(x, pv, ev, pbins, ebins, ptab, etab)


def _pred_call(x, x1, mk, wd, wp_, we):
    f32 = jnp.float32
    in_specs = [
        _batch_spec((_T, _D)),
        _batch_spec((_T, _D)),
        _batch_spec((_T, 1)),
    ]
    args = [x, x1, mk]
    for w1, b1, w2, b2, wpb in (wd, wp_, we):
        in_specs += [_full_spec((3 * _D, _FS)), _full_spec((1, _FS)),
                     _full_spec((3 * _FS, _FS)), _full_spec((1, _FS)),
                     _full_spec((2, _FS))]
        args += [w1, b1, w2, b2, wpb]
    out_specs = [
        pl.BlockSpec((1, _T, 1), lambda i: (i, 0, 0)),
        pl.BlockSpec((1, _T, 1), lambda i: (i, 0, 0)),
        pl.BlockSpec((1, _T, 1), lambda i: (i, 0, 0)),
    ]
    out_shape = [
        jax.ShapeDtypeStruct((_B, _T, 1), f32),
        jax.ShapeDtypeStruct((_B, _T, 1), f32),
        jax.ShapeDtypeStruct((_B, _T, 1), f32),
    ]
    return pl.pallas_call(
        _pred_body,
        grid=(_B,),
        in_specs=in_specs,
        out_specs=out_specs,
        out_shape=out_shape,
    )(*args)


def _sc_body(dur_hbm, x2_hbm, out_hbm, lens_hbm, dur_v, a_v, gidx_v, buf_v,
             gidx88_v, buf88_v, gidx4_v, buf4_v, lens_v, sem):
    c = lax.axis_index("c")
    s = lax.axis_index("s")
    wid = s * 2 + c
    b = wid // 2
    h = wid % 2

    pltpu.sync_copy(dur_hbm.at[b], dur_v)

    iota = lax.broadcasted_iota(jnp.int32, (16,), 0)
    zeros16 = jnp.zeros((16,), jnp.int32)

    def zbody(j, carry):
        a_v[pl.ds(j * 16, 16)] = zeros16
        return carry
    lax.fori_loop(0, _PAD_A // 16, zbody, jnp.int32(0))

    def cbody(i, carry):
        d = dur_v[0, pl.ds(i * 16, 16)]
        cum = plsc.cumsum(d) + carry
        prev = cum - d
        m = (d > 0) & (prev < _PAD_A)
        plsc.store_scatter(a_v, [prev], iota + i * 16, mask=m)
        return jnp.max(cum)
    total = lax.fori_loop(0, _T // 16, cbody, jnp.int32(0))
    len_b = jnp.minimum(total, _MAXLEN)

    @pl.when(h == 0)
    def _():
        lens_v[0, :] = jnp.full((16,), total, jnp.int32)
        pltpu.sync_copy(lens_v, lens_hbm.at[b])

    def fbody(j, carry):
        m = jnp.maximum(plsc.cummax(a_v[pl.ds(j * 16, 16)]), carry)
        a_v[pl.ds(j * 16, 16)] = m
        return jnp.max(m)
    lax.fori_loop(0, _PAD_A // 16, fbody, jnp.int32(0))

    def fill_idx(idx_ref, off, t0):
        t = t0 + off
        a = a_v[pl.ds(t, 16)]
        g = jnp.where(iota + t < len_b, b * _T + a, _ZROW)
        idx_ref[pl.ds(off, 16)] = g

    for hh in (0, 1):
        @pl.when(h == hh)
        def _(hh=hh):
            for t0, ln in _CHUNKS[hh]:
                if ln == 128:
                    for j in range(8):
                        fill_idx(gidx_v, j * 16, t0)
                    pltpu.async_copy(x2_hbm.at[gidx_v], buf_v, sem).wait()
                    pltpu.sync_copy(buf_v, out_hbm.at[b, pl.ds(t0, 128)])
                else:
                    for off in (0, 16, 32, 48, 64, ln - 16):
                        fill_idx(gidx88_v, off, t0)
                    pltpu.async_copy(x2_hbm.at[gidx88_v], buf88_v, sem).wait()
                    pltpu.sync_copy(buf88_v, out_hbm.at[b, pl.ds(t0, ln)])
            if hh == 0:
                return
            t4 = _MAXLEN - 4
            a4 = a_v[pl.ds(_PAD_A - 16, 16)]
            g4 = jnp.where(iota + (_PAD_A - 16) < len_b, b * _T + a4, _ZROW)
            plsc.store_scatter(gidx4_v, [iota - 8], g4,
                               mask=(iota >= 8) & (iota < 12))
            pltpu.async_copy(x2_hbm.at[gidx4_v], buf4_v, sem).wait()
            pltpu.sync_copy(buf4_v, out_hbm.at[b, pl.ds(t4, 4)])


@functools.cache
def _sc_call():
    return pl.kernel(
        _sc_body,
        out_type=[jax.ShapeDtypeStruct((_B, _MAXLEN, _D), jnp.float32),
                  jax.ShapeDtypeStruct((_B, 1, 16), jnp.int32)],
        mesh=plsc.VectorSubcoreMesh(core_axis_name="c", subcore_axis_name="s",
                                    num_cores=2, num_subcores=16),
        scratch_types=[pltpu.VMEM((1, _T), jnp.int32),
                       pltpu.VMEM((_PAD_A,), jnp.int32),
                       pltpu.VMEM((128,), jnp.int32),
                       pltpu.VMEM((128, _D), jnp.float32),
                       pltpu.VMEM((88,), jnp.int32),
                       pltpu.VMEM((88, _D), jnp.float32),
                       pltpu.VMEM((4,), jnp.int32),
                       pltpu.VMEM((4, _D), jnp.float32),
                       pltpu.VMEM((1, 16), jnp.int32),
                       pltpu.SemaphoreType.DMA],
        compiler_params=pltpu.CompilerParams(needs_layout_passes=False),
    )


def _fold_bn(p):
    s1 = p['g1'] / jnp.sqrt(p['v1'] + 1e-5)
    s2 = p['g2'] / jnp.sqrt(p['v2'] + 1e-5)
    w1 = p['w1'] * s1[:, None, None]
    w2 = p['w2'] * s2[:, None, None]
    W1 = jnp.concatenate([w1[:, :, 0].T, w1[:, :, 1].T, w1[:, :, 2].T], axis=0)
    W2 = jnp.concatenate([w2[:, :, 0].T, w2[:, :, 1].T, w2[:, :, 2].T], axis=0)
    b1 = ((p['b1'] - p['m1']) * s1 + p['be1'])[None, :]
    b2 = ((p['b2'] - p['m2']) * s2 + p['be2'])[None, :]
    wpb = jnp.concatenate([p['wp'], jnp.full((1, _FS), p['bp'][0])], axis=0)
    return W1, b1, W2, b2, wpb


def _pad_bins(bins):
    return jnp.concatenate([bins, jnp.full((1,), 1e30, jnp.float32)])[None, :]


def kernel(x, src_mask, src_lengths, durations, pitch, energy, max_length,
           params, p_control=1.0, e_control=1.0, d_control=1.0):
    f32 = jnp.float32
    pv = pitch[..., None].astype(f32)
    ev = energy[..., None].astype(f32)
    mk = src_mask[..., None].astype(f32)
    wd = _fold_bn(params['dur'])
    wp_ = _fold_bn(params['pitch'])
    we = _fold_bn(params['energy'])

    x1, x2_pad = _emb_call(
        x, pv, ev,
        _pad_bins(params['pitch_bins']), _pad_bins(params['energy_bins']),
        params['pitch_table'], params['energy_table'])

    dur_rounded = jnp.round(durations.astype(f32) * d_control).astype(jnp.int32)
    out, lens16 = _sc_call()(dur_rounded.reshape(_B, 1, _T), x2_pad)

    ld_pad, pp_pad, ep_pad = _pred_call(x, x1, mk, wd, wp_, we)
    log_d = ld_pad[:, :, 0]
    p_pred = pp_pad[:, :, 0]
    e_pred = ep_pad[:, :, 0]
    mel_lens = jnp.minimum(lens16[:, 0, 0], max_length)
    mel_mask = jnp.arange(_MAXLEN)[None, :] >= mel_lens[:, None]
    return out, log_d, p_pred, e_pred, mel_lens, mel_mask
